# Initial kernel scaffold; baseline (speedup 1.0000x reference)
import jax
import jax.numpy as jnp
from jax.experimental import pallas as pl


def kernel(h, edge_index, Q0, qb0, W0, wb0, Q1, qb1, W1, wb1, G, Gb, g, bn_out_gamma, bn_out_beta, bn_gamma, bn_beta):
    raise NotImplementedError("write your pallas kernel here")


# speedup vs baseline: 8.1771x; 8.1771x over previous
import functools

import jax
import jax.numpy as jnp
from jax import lax
from jax.experimental import pallas as pl
from jax.experimental.pallas import tpu as pltpu
from jax.experimental.pallas import tpu_sc as plsc

NC = 2
NS = 16
NW = NC * NS

G = 125
ROWS_PER_LOAD = 8


def _sc_aggregate(h, src2d, dst2d, dst1d, zeros_gd, with_deg):
  n_nodes, d = h.shape
  n_rows = src2d.shape[0]
  rows_per_tile = n_rows // NW
  n_loads = rows_per_tile // ROWS_PER_LOAD
  epl = ROWS_PER_LOAD * G
  chz = 104
  n_chunks = n_nodes // (chz * NS)
  npt = chz * n_chunks
  rem = n_nodes - npt * NS
  assert rows_per_tile % ROWS_PER_LOAD == 0
  assert rem % 8 == 0 and rem <= chz

  mesh = plsc.VectorSubcoreMesh(core_axis_name="c", subcore_axis_name="s")

  out_type = [jax.ShapeDtypeStruct((NC, n_nodes, d), jnp.float32)]
  if with_deg:
    out_type.append(jax.ShapeDtypeStruct((NW, 1, n_nodes), jnp.float32))

  scratch = [
      pltpu.VMEM((ROWS_PER_LOAD, G), jnp.int32),
      pltpu.VMEM((ROWS_PER_LOAD, G), jnp.int32),
      pltpu.VMEM((G, d), jnp.float32),
      pltpu.VMEM((chz, d), jnp.float32),
      pltpu.VMEM_SHARED((n_nodes, d), jnp.float32),
      pltpu.SemaphoreType.DMA,
  ]
  if with_deg:
    scratch += [
        pltpu.VMEM((epl,), jnp.int32),
        pltpu.VMEM((n_nodes,), jnp.float32),
    ]

  @functools.partial(
      pl.kernel, mesh=mesh, out_type=out_type, scratch_types=scratch,
      compiler_params=pltpu.CompilerParams(needs_layout_passes=False))
  def agg_kernel(h_hbm, src_hbm, dst_hbm, dstf_hbm, zgd_hbm, *rest):
    if with_deg:
      agg_out, deg_out = rest[0], rest[1]
      (src_v, dst_v, rows_v, zero_v, acc_sh, sem, dstf_v, deg_v) = rest[2:]
    else:
      agg_out = rest[0]
      (src_v, dst_v, rows_v, zero_v, acc_sh, sem) = rest[1:]

    cid = lax.axis_index("c")
    sid = lax.axis_index("s")

    pltpu.sync_copy(zgd_hbm, zero_v)
    for k in range(n_chunks):
      pltpu.sync_copy(zero_v, acc_sh.at[pl.ds(sid * npt + k * chz, chz)])

    @pl.when(sid == NS - 1)
    def _zero_tail():
      pltpu.sync_copy(zero_v.at[pl.ds(0, rem)],
                      acc_sh.at[pl.ds(NS * npt, rem)])

    if with_deg:
      def zfill(i, _):
        deg_v[pl.ds(i * 16, 16)] = jnp.zeros((16,), jnp.float32)
        return 0
      lax.fori_loop(0, n_nodes // 16, zfill, 0, unroll=False)
    plsc.subcore_barrier()

    wid = sid * NC + cid
    row0 = wid * rows_per_tile

    ones16 = jnp.ones((16,), jnp.float32)
    tail_mask = lax.iota(jnp.int32, 16) >= (16 - epl % 16 if epl % 16 else 16)

    def body(ld, _):
      base = row0 + ld * ROWS_PER_LOAD
      pltpu.sync_copy(src_hbm.at[pl.ds(base, ROWS_PER_LOAD)], src_v)
      pltpu.sync_copy(dst_hbm.at[pl.ds(base, ROWS_PER_LOAD)], dst_v)
      if with_deg:
        pltpu.sync_copy(dstf_hbm.at[pl.ds(base * G, epl)], dstf_v)

        def degbody(k, _):
          idx = dstf_v[pl.ds(k * 16, 16)]
          plsc.addupdate_scatter(deg_v, [idx], ones16)
          return 0
        lax.fori_loop(0, epl // 16, degbody, 0, unroll=False)
        if epl % 16:
          idx = dstf_v[pl.ds(epl - 16, 16)]
          plsc.addupdate_scatter(deg_v, [idx], ones16, mask=tail_mask)
      for j in range(ROWS_PER_LOAD):
        pltpu.async_copy(h_hbm.at[src_v.at[j]], rows_v, sem).wait()
        pltpu.sync_copy(rows_v, acc_sh.at[dst_v.at[j]], add=True)
      return 0
    lax.fori_loop(0, n_loads, body, 0, unroll=False)

    plsc.subcore_barrier()

    pltpu.sync_copy(acc_sh.at[pl.ds(sid * npt, npt)],
                    agg_out.at[cid, pl.ds(sid * npt, npt)])

    @pl.when(sid == NS - 1)
    def _dump_tail():
      pltpu.sync_copy(acc_sh.at[pl.ds(NS * npt, rem)],
                      agg_out.at[cid, pl.ds(NS * npt, rem)])

    if with_deg:
      pltpu.sync_copy(deg_v, deg_out.at[wid, 0])

  return agg_kernel(h, src2d, dst2d, dst1d, zeros_gd)


def _tc_layer(aggp, degp, h, q, qb, w, wb):
  n_nodes, d = h.shape
  bn = 1000
  grid = n_nodes // bn

  def body(aggp_ref, degp_ref, h_ref, q_ref, qb_ref, w_ref, wb_ref, o_ref):
    agg = aggp_ref[0] + aggp_ref[1]
    deg = jnp.sum(degp_ref[:, 0, 0, :], axis=0)
    deg = jnp.maximum(deg, 1.0)[:, None]
    agg = agg / deg
    nh = jnp.maximum(
        lax.dot_general(agg, q_ref[...], (((1,), (0,)), ((), ())),
                        preferred_element_type=jnp.float32) + qb_ref[...],
        0.0)
    z = (lax.dot_general(h_ref[...], w_ref[pl.ds(0, d), :],
                         (((1,), (0,)), ((), ())),
                         preferred_element_type=jnp.float32)
         + lax.dot_general(nh, w_ref[pl.ds(d, d), :],
                           (((1,), (0,)), ((), ())),
                           preferred_element_type=jnp.float32)
         + wb_ref[...])
    z = jnp.maximum(z, 0.0)
    nrm = jnp.sqrt(jnp.sum(z * z, axis=1, keepdims=True))
    o_ref[...] = z / jnp.maximum(nrm, 1e-8)

  return pl.pallas_call(
      body,
      grid=(grid,),
      in_specs=[
          pl.BlockSpec((NC, bn, d), lambda i: (0, i, 0)),
          pl.BlockSpec((NW, 1, 1, bn), lambda i: (0, i, 0, 0)),
          pl.BlockSpec((bn, d), lambda i: (i, 0)),
          pl.BlockSpec((d, d), lambda i: (0, 0)),
          pl.BlockSpec((d,), lambda i: (0,)),
          pl.BlockSpec((2 * d, d), lambda i: (0, 0)),
          pl.BlockSpec((d,), lambda i: (0,)),
      ],
      out_specs=pl.BlockSpec((bn, d), lambda i: (i, 0)),
      out_shape=jax.ShapeDtypeStruct((n_nodes, d), jnp.float32),
  )(aggp, degp.reshape(NW, grid, 1, bn), h, q, qb, w, wb)


def _tc_head(h2, g_mat, gb, gam1, bet1, gam2, bet2):
  n_nodes, d = h2.shape
  bn = 1000
  grid = n_nodes // bn

  def body(h2_ref, gm_ref, gb_ref, g1_ref, b1_ref, g2_ref, b2_ref,
           o_ref, zs_ref):
    i = pl.program_id(0)
    z = jnp.maximum(
        lax.dot_general(h2_ref[...], gm_ref[...], (((1,), (0,)), ((), ())),
                        preferred_element_type=jnp.float32) + gb_ref[...],
        0.0)
    zs_ref[pl.ds(i * bn, bn), :] = z

    @pl.when(i == grid - 1)
    def _():
      zz = zs_ref[...]
      inv_n = 1.0 / n_nodes
      mu = jnp.sum(zz, axis=0) * inv_n
      c = zz - mu
      var = jnp.sum(c * c, axis=0) * inv_n
      y = g1_ref[...] * c / jnp.sqrt(var + 1e-5) + b1_ref[...]
      mu2 = jnp.sum(y, axis=0) * inv_n
      c2 = y - mu2
      var2 = jnp.sum(c2 * c2, axis=0) * inv_n
      o_ref[...] = g2_ref[...] * c2 / jnp.sqrt(var2 + 1e-5) + b2_ref[...]

  return pl.pallas_call(
      body,
      grid=(grid,),
      in_specs=[
          pl.BlockSpec((bn, d), lambda i: (i, 0)),
          pl.BlockSpec((d, d), lambda i: (0, 0)),
          pl.BlockSpec((d,), lambda i: (0,)),
          pl.BlockSpec((d,), lambda i: (0,)),
          pl.BlockSpec((d,), lambda i: (0,)),
          pl.BlockSpec((d,), lambda i: (0,)),
          pl.BlockSpec((d,), lambda i: (0,)),
      ],
      out_specs=pl.BlockSpec((n_nodes, d), lambda i: (0, 0)),
      out_shape=jax.ShapeDtypeStruct((n_nodes, d), jnp.float32),
      scratch_shapes=[pltpu.VMEM((n_nodes, d), jnp.float32)],
  )(h2, g_mat, gb, gam1, bet1, gam2, bet2)


def kernel(h, edge_index, Q0, qb0, W0, wb0, Q1, qb1, W1, wb1, G_mat, Gb, g,
           bn_out_gamma, bn_out_beta, bn_gamma, bn_beta):
  e = edge_index.shape[1]
  d = h.shape[1]
  src2d = edge_index[0].reshape(e // G, G)
  dst1d = edge_index[1]
  dst2d = dst1d.reshape(e // G, G)
  zeros_gd = jnp.zeros((104, d), jnp.float32)

  agg1p, degp = _sc_aggregate(h, src2d, dst2d, dst1d, zeros_gd,
                              with_deg=True)
  h1 = _tc_layer(agg1p, degp, h, Q0, qb0, W0, wb0)
  (agg2p,) = _sc_aggregate(h1, src2d, dst2d, dst1d, zeros_gd,
                           with_deg=False)
  h2 = _tc_layer(agg2p, degp, h1, Q1, qb1, W1, wb1)
  gam1 = g * bn_out_gamma
  bet1 = g * bn_out_beta
  return _tc_head(h2, G_mat, Gb, gam1, bet1, bn_gamma, bn_beta)


# speedup vs baseline: 11.1369x; 1.3620x over previous
import functools

import jax
import jax.numpy as jnp
from jax import lax
from jax.experimental import pallas as pl
from jax.experimental.pallas import tpu as pltpu
from jax.experimental.pallas import tpu_sc as plsc

NC = 2
NS = 16
NW = NC * NS

G = 125
ROWS_PER_LOAD = 8


def _sc_aggregate(h, src2d, dst2d, dst1d, zeros_gd, with_deg):
  n_nodes, d = h.shape
  n_rows = src2d.shape[0]
  rows_per_tile = n_rows // NW
  n_loads = rows_per_tile // ROWS_PER_LOAD
  epl = ROWS_PER_LOAD * G
  npt = (n_nodes // NS) // 8 * 8
  rem = n_nodes - npt * NS
  assert rows_per_tile % ROWS_PER_LOAD == 0
  assert rem % 8 == 0

  mesh = plsc.VectorSubcoreMesh(core_axis_name="c", subcore_axis_name="s")

  out_type = [jax.ShapeDtypeStruct((NC, n_nodes, d), jnp.float32)]
  if with_deg:
    out_type.append(jax.ShapeDtypeStruct((NW, 1, n_nodes), jnp.float32))

  scratch = [
      pltpu.VMEM((ROWS_PER_LOAD, G), jnp.int32),
      pltpu.VMEM((ROWS_PER_LOAD, G), jnp.int32),
      pltpu.VMEM((G, d), jnp.float32),
      pltpu.VMEM((G, d), jnp.float32),
      pltpu.VMEM_SHARED((n_nodes, d), jnp.float32),
      pltpu.SemaphoreType.DMA,
      pltpu.SemaphoreType.DMA,
  ]
  if with_deg:
    scratch += [
        pltpu.VMEM((epl,), jnp.int32),
        pltpu.VMEM((n_nodes,), jnp.float32),
    ]

  @functools.partial(
      pl.kernel, mesh=mesh, out_type=out_type, scratch_types=scratch,
      compiler_params=pltpu.CompilerParams(needs_layout_passes=False))
  def agg_kernel(h_hbm, src_hbm, dst_hbm, dstf_hbm, zgd_hbm, *rest):
    if with_deg:
      agg_out, deg_out = rest[0], rest[1]
      (src_v, dst_v, rows_a, rows_b, acc_sh, sem_a, sem_b,
       dstf_v, deg_v) = rest[2:]
    else:
      agg_out = rest[0]
      (src_v, dst_v, rows_a, rows_b, acc_sh, sem_a, sem_b) = rest[1:]

    cid = lax.axis_index("c")
    sid = lax.axis_index("s")

    @pl.when(sid < NS - 1)
    def _zero_main():
      pltpu.sync_copy(zgd_hbm.at[pl.ds(0, npt)],
                      acc_sh.at[pl.ds(sid * npt, npt)])

    @pl.when(sid == NS - 1)
    def _zero_tail():
      pltpu.sync_copy(zgd_hbm.at[pl.ds(0, npt + rem)],
                      acc_sh.at[pl.ds(sid * npt, npt + rem)])

    if with_deg:
      def zfill(i, _):
        deg_v[pl.ds(i * 16, 16)] = jnp.zeros((16,), jnp.float32)
        return 0
      lax.fori_loop(0, n_nodes // 16, zfill, 0, unroll=False)
    plsc.subcore_barrier()

    wid = sid * NC + cid
    row0 = wid * rows_per_tile

    ones16 = jnp.ones((16,), jnp.float32)
    tail_mask = lax.iota(jnp.int32, 16) >= (16 - epl % 16 if epl % 16 else 16)

    bufs = (rows_a, rows_b)
    sems = (sem_a, sem_b)

    def body(ld, _):
      base = row0 + ld * ROWS_PER_LOAD
      pltpu.sync_copy(src_hbm.at[pl.ds(base, ROWS_PER_LOAD)], src_v)
      pltpu.sync_copy(dst_hbm.at[pl.ds(base, ROWS_PER_LOAD)], dst_v)
      d_prev = pltpu.async_copy(h_hbm.at[src_v.at[0]], bufs[0], sems[0])
      if with_deg:
        pltpu.sync_copy(dstf_hbm.at[pl.ds(base * G, epl)], dstf_v)

        def degbody(k, _):
          idx = dstf_v[pl.ds(k * 16, 16)]
          plsc.addupdate_scatter(deg_v, [idx], ones16)
          return 0
        lax.fori_loop(0, epl // 16, degbody, 0, unroll=False)
        if epl % 16:
          idx = dstf_v[pl.ds(epl - 16, 16)]
          plsc.addupdate_scatter(deg_v, [idx], ones16, mask=tail_mask)
      for j in range(ROWS_PER_LOAD):
        if j + 1 < ROWS_PER_LOAD:
          d_next = pltpu.async_copy(h_hbm.at[src_v.at[j + 1]],
                                    bufs[(j + 1) % 2], sems[(j + 1) % 2])
        d_prev.wait()
        pltpu.sync_copy(bufs[j % 2], acc_sh.at[dst_v.at[j]], add=True)
        if j + 1 < ROWS_PER_LOAD:
          d_prev = d_next
      return 0
    lax.fori_loop(0, n_loads, body, 0, unroll=False)

    plsc.subcore_barrier()

    pltpu.sync_copy(acc_sh.at[pl.ds(sid * npt, npt)],
                    agg_out.at[cid, pl.ds(sid * npt, npt)])

    @pl.when(sid == NS - 1)
    def _dump_tail():
      pltpu.sync_copy(acc_sh.at[pl.ds(NS * npt, rem)],
                      agg_out.at[cid, pl.ds(NS * npt, rem)])

    if with_deg:
      pltpu.sync_copy(deg_v, deg_out.at[wid, 0])

  return agg_kernel(h, src2d, dst2d, dst1d, zeros_gd)


def _tc_layer(aggp, degp, h, q, qb, w, wb):
  n_nodes, d = h.shape
  bn = 1000
  grid = n_nodes // bn

  def body(aggp_ref, degp_ref, h_ref, q_ref, qb_ref, w_ref, wb_ref, o_ref):
    agg = aggp_ref[0] + aggp_ref[1]
    deg = jnp.sum(degp_ref[:, 0, 0, :], axis=0)
    deg = jnp.maximum(deg, 1.0)[:, None]
    agg = agg / deg
    nh = jnp.maximum(
        lax.dot_general(agg, q_ref[...], (((1,), (0,)), ((), ())),
                        preferred_element_type=jnp.float32) + qb_ref[...],
        0.0)
    z = (lax.dot_general(h_ref[...], w_ref[pl.ds(0, d), :],
                         (((1,), (0,)), ((), ())),
                         preferred_element_type=jnp.float32)
         + lax.dot_general(nh, w_ref[pl.ds(d, d), :],
                           (((1,), (0,)), ((), ())),
                           preferred_element_type=jnp.float32)
         + wb_ref[...])
    z = jnp.maximum(z, 0.0)
    nrm = jnp.sqrt(jnp.sum(z * z, axis=1, keepdims=True))
    o_ref[...] = z / jnp.maximum(nrm, 1e-8)

  return pl.pallas_call(
      body,
      grid=(grid,),
      in_specs=[
          pl.BlockSpec((NC, bn, d), lambda i: (0, i, 0)),
          pl.BlockSpec((NW, 1, 1, bn), lambda i: (0, i, 0, 0)),
          pl.BlockSpec((bn, d), lambda i: (i, 0)),
          pl.BlockSpec((d, d), lambda i: (0, 0)),
          pl.BlockSpec((d,), lambda i: (0,)),
          pl.BlockSpec((2 * d, d), lambda i: (0, 0)),
          pl.BlockSpec((d,), lambda i: (0,)),
      ],
      out_specs=pl.BlockSpec((bn, d), lambda i: (i, 0)),
      out_shape=jax.ShapeDtypeStruct((n_nodes, d), jnp.float32),
  )(aggp, degp.reshape(NW, grid, 1, bn), h, q, qb, w, wb)


def _tc_head(h2, g_mat, gb, gam1, bet1, gam2, bet2):
  n_nodes, d = h2.shape
  bn = 1000
  grid = n_nodes // bn

  def body(h2_ref, gm_ref, gb_ref, g1_ref, b1_ref, g2_ref, b2_ref,
           o_ref, zs_ref):
    i = pl.program_id(0)
    z = jnp.maximum(
        lax.dot_general(h2_ref[...], gm_ref[...], (((1,), (0,)), ((), ())),
                        preferred_element_type=jnp.float32) + gb_ref[...],
        0.0)
    zs_ref[pl.ds(i * bn, bn), :] = z

    @pl.when(i == grid - 1)
    def _():
      zz = zs_ref[...]
      inv_n = 1.0 / n_nodes
      mu = jnp.sum(zz, axis=0) * inv_n
      c = zz - mu
      var = jnp.sum(c * c, axis=0) * inv_n
      y = g1_ref[...] * c / jnp.sqrt(var + 1e-5) + b1_ref[...]
      mu2 = jnp.sum(y, axis=0) * inv_n
      c2 = y - mu2
      var2 = jnp.sum(c2 * c2, axis=0) * inv_n
      o_ref[...] = g2_ref[...] * c2 / jnp.sqrt(var2 + 1e-5) + b2_ref[...]

  return pl.pallas_call(
      body,
      grid=(grid,),
      in_specs=[
          pl.BlockSpec((bn, d), lambda i: (i, 0)),
          pl.BlockSpec((d, d), lambda i: (0, 0)),
          pl.BlockSpec((d,), lambda i: (0,)),
          pl.BlockSpec((d,), lambda i: (0,)),
          pl.BlockSpec((d,), lambda i: (0,)),
          pl.BlockSpec((d,), lambda i: (0,)),
          pl.BlockSpec((d,), lambda i: (0,)),
      ],
      out_specs=pl.BlockSpec((n_nodes, d), lambda i: (0, 0)),
      out_shape=jax.ShapeDtypeStruct((n_nodes, d), jnp.float32),
      scratch_shapes=[pltpu.VMEM((n_nodes, d), jnp.float32)],
  )(h2, g_mat, gb, gam1, bet1, gam2, bet2)


def kernel(h, edge_index, Q0, qb0, W0, wb0, Q1, qb1, W1, wb1, G_mat, Gb, g,
           bn_out_gamma, bn_out_beta, bn_gamma, bn_beta):
  e = edge_index.shape[1]
  d = h.shape[1]
  src2d = edge_index[0].reshape(e // G, G)
  dst1d = edge_index[1]
  dst2d = dst1d.reshape(e // G, G)
  zeros_gd = jnp.zeros((640, d), jnp.float32)

  agg1p, degp = _sc_aggregate(h, src2d, dst2d, dst1d, zeros_gd,
                              with_deg=True)
  h1 = _tc_layer(agg1p, degp, h, Q0, qb0, W0, wb0)
  (agg2p,) = _sc_aggregate(h1, src2d, dst2d, dst1d, zeros_gd,
                           with_deg=False)
  h2 = _tc_layer(agg2p, degp, h1, Q1, qb1, W1, wb1)
  gam1 = g * bn_out_gamma
  bet1 = g * bn_out_beta
  return _tc_head(h2, G_mat, Gb, gam1, bet1, bn_gamma, bn_beta)


# speedup vs baseline: 12.3568x; 1.1095x over previous
import functools

import jax
import jax.numpy as jnp
from jax import lax
from jax.experimental import pallas as pl
from jax.experimental.pallas import tpu as pltpu
from jax.experimental.pallas import tpu_sc as plsc

NC = 2
NS = 16
NW = NC * NS

G = 125
RPL = 8


def _sc_aggregate(h, edges3d, dst1d, zeros_gd, with_deg):
  n_nodes, d = h.shape
  n_rows = edges3d.shape[0]
  rows_per_tile = n_rows // NW
  n_loads = rows_per_tile // RPL
  n_pairs = n_loads // 2
  epl = RPL * G
  npt = (n_nodes // NS) // 8 * 8
  rem = n_nodes - npt * NS
  assert rows_per_tile % RPL == 0 and n_loads % 2 == 0
  assert rem % 8 == 0

  mesh = plsc.VectorSubcoreMesh(core_axis_name="c", subcore_axis_name="s")

  out_type = [jax.ShapeDtypeStruct((NC, n_nodes, d), jnp.float32)]
  if with_deg:
    out_type.append(jax.ShapeDtypeStruct((NW, 1, n_nodes), jnp.float32))

  scratch = [
      pltpu.VMEM((RPL, 2, G), jnp.int32),
      pltpu.VMEM((RPL, 2, G), jnp.int32),
      pltpu.VMEM((G, d), jnp.float32),
      pltpu.VMEM((G, d), jnp.float32),
      pltpu.VMEM_SHARED((n_nodes, d), jnp.float32),
      pltpu.SemaphoreType.DMA,
      pltpu.SemaphoreType.DMA,
      pltpu.SemaphoreType.DMA,
  ]
  if with_deg:
    scratch += [
        pltpu.VMEM((epl,), jnp.int32),
        pltpu.VMEM((epl,), jnp.int32),
        pltpu.VMEM((n_nodes,), jnp.float32),
    ]

  @functools.partial(
      pl.kernel, mesh=mesh, out_type=out_type, scratch_types=scratch,
      compiler_params=pltpu.CompilerParams(needs_layout_passes=False))
  def agg_kernel(h_hbm, edg_hbm, dstf_hbm, zgd_hbm, *rest):
    if with_deg:
      agg_out, deg_out = rest[0], rest[1]
      (edg_a, edg_b, rows_a, rows_b, acc_sh, sem_a, sem_b, sem_i,
       dstf_a, dstf_b, deg_v) = rest[2:]
    else:
      agg_out = rest[0]
      (edg_a, edg_b, rows_a, rows_b, acc_sh, sem_a, sem_b, sem_i) = rest[1:]
      dstf_a = dstf_b = deg_v = None

    cid = lax.axis_index("c")
    sid = lax.axis_index("s")
    wid = sid * NC + cid
    row0 = wid * rows_per_tile

    bufs = (rows_a, rows_b)
    sems = (sem_a, sem_b)
    edgs = (edg_a, edg_b)
    dstfs = (dstf_a, dstf_b)

    def idx_start(half, load):
      pltpu.async_copy(edg_hbm.at[pl.ds(row0 + load * RPL, RPL)],
                       edgs[half], sem_i)
      if with_deg:
        pltpu.async_copy(dstf_hbm.at[pl.ds((row0 + load * RPL) * G, epl)],
                         dstfs[half], sem_i)

    def idx_wait(half):
      pltpu.make_async_copy(edg_hbm.at[pl.ds(0, RPL)], edgs[half],
                            sem_i).wait()
      if with_deg:
        pltpu.make_async_copy(dstf_hbm.at[pl.ds(0, epl)], dstfs[half],
                              sem_i).wait()

    def gather_start(half, j, buf):
      pltpu.async_copy(h_hbm.at[edgs[half].at[j, 0]], bufs[buf],
                       sems[buf])

    def gather_wait(half, j, buf):
      pltpu.make_async_copy(h_hbm.at[edgs[half].at[j, 0]], bufs[buf],
                            sems[buf]).wait()

    ones16 = jnp.ones((16,), jnp.float32)
    tail = epl % 16
    tail_mask = lax.iota(jnp.int32, 16) >= (16 - tail)

    def deg_accumulate(half):
      dstf = dstfs[half]

      def degbody(k, _):
        plsc.addupdate_scatter(deg_v, [dstf[pl.ds(k * 16, 16)]], ones16)
        return 0
      lax.fori_loop(0, epl // 16, degbody, 0, unroll=False)
      if tail:
        plsc.addupdate_scatter(deg_v, [dstf[pl.ds(epl - 16, 16)]], ones16,
                               mask=tail_mask)

    def process(half, cross_issue):
      if with_deg:
        deg_accumulate(half)
      for j in range(RPL):
        if j + 1 < RPL:
          gather_start(half, j + 1, (j + 1) % 2)
        else:
          cross_issue()
        gather_wait(half, j, j % 2)
        pltpu.sync_copy(bufs[j % 2], acc_sh.at[edgs[half].at[j, 1]],
                        add=True)

    idx_start(0, 0)

    @pl.when(sid < NS - 1)
    def _zero_main():
      pltpu.sync_copy(zgd_hbm.at[pl.ds(0, npt)],
                      acc_sh.at[pl.ds(sid * npt, npt)])

    @pl.when(sid == NS - 1)
    def _zero_tail():
      pltpu.sync_copy(zgd_hbm.at[pl.ds(0, npt + rem)],
                      acc_sh.at[pl.ds(sid * npt, npt + rem)])

    if with_deg:
      def zfill(i, _):
        deg_v[pl.ds(i * 16, 16)] = jnp.zeros((16,), jnp.float32)
        return 0
      lax.fori_loop(0, n_nodes // 16, zfill, 0, unroll=False)

    idx_wait(0)
    gather_start(0, 0, 0)
    plsc.subcore_barrier()

    def body(k, _):
      idx_start(1, 2 * k + 1)

      def cross_ab():
        idx_wait(1)
        gather_start(1, 0, 0)
      process(0, cross_ab)

      @pl.when(k < n_pairs - 1)
      def _prefetch_a():
        idx_start(0, 2 * k + 2)

      def cross_ba():
        @pl.when(k < n_pairs - 1)
        def _():
          idx_wait(0)
          gather_start(0, 0, 0)
      process(1, cross_ba)
      return 0
    lax.fori_loop(0, n_pairs, body, 0, unroll=False)

    plsc.subcore_barrier()

    pltpu.sync_copy(acc_sh.at[pl.ds(sid * npt, npt)],
                    agg_out.at[cid, pl.ds(sid * npt, npt)])

    @pl.when(sid == NS - 1)
    def _dump_tail():
      pltpu.sync_copy(acc_sh.at[pl.ds(NS * npt, rem)],
                      agg_out.at[cid, pl.ds(NS * npt, rem)])

    if with_deg:
      pltpu.sync_copy(deg_v, deg_out.at[wid, 0])

  return agg_kernel(h, edges3d, dst1d, zeros_gd)


def _split_matmul(x, nh, w, wb, d):
  z = (lax.dot_general(x, w[pl.ds(0, d), :], (((1,), (0,)), ((), ())),
                       preferred_element_type=jnp.float32)
       + lax.dot_general(nh, w[pl.ds(d, d), :], (((1,), (0,)), ((), ())),
                         preferred_element_type=jnp.float32)
       + wb[...])
  return jnp.maximum(z, 0.0)


def _pinconv_block(aggp_ref, degp_ref, x_ref, q_ref, qb_ref, w_ref, wb_ref,
                   d):
  agg = aggp_ref[0] + aggp_ref[1]
  deg = jnp.sum(degp_ref[:, 0, 0, :], axis=0)
  deg = jnp.maximum(deg, 1.0)[:, None]
  agg = agg / deg
  nh = jnp.maximum(
      lax.dot_general(agg, q_ref[...], (((1,), (0,)), ((), ())),
                      preferred_element_type=jnp.float32) + qb_ref[...],
      0.0)
  z = _split_matmul(x_ref[...], nh, w_ref, wb_ref, d)
  nrm = jnp.sqrt(jnp.sum(z * z, axis=1, keepdims=True))
  return z / jnp.maximum(nrm, 1e-8)


def _tc_layer(aggp, degp, h, q, qb, w, wb):
  n_nodes, d = h.shape
  bn = 1000
  grid = n_nodes // bn

  def body(aggp_ref, degp_ref, h_ref, q_ref, qb_ref, w_ref, wb_ref, o_ref):
    o_ref[...] = _pinconv_block(aggp_ref, degp_ref, h_ref, q_ref, qb_ref,
                                w_ref, wb_ref, d)

  return pl.pallas_call(
      body,
      grid=(grid,),
      in_specs=[
          pl.BlockSpec((NC, bn, d), lambda i: (0, i, 0)),
          pl.BlockSpec((NW, 1, 1, bn), lambda i: (0, i, 0, 0)),
          pl.BlockSpec((bn, d), lambda i: (i, 0)),
          pl.BlockSpec((d, d), lambda i: (0, 0)),
          pl.BlockSpec((d,), lambda i: (0,)),
          pl.BlockSpec((2 * d, d), lambda i: (0, 0)),
          pl.BlockSpec((d,), lambda i: (0,)),
      ],
      out_specs=pl.BlockSpec((bn, d), lambda i: (i, 0)),
      out_shape=jax.ShapeDtypeStruct((n_nodes, d), jnp.float32),
  )(aggp, degp.reshape(NW, grid, 1, bn), h, q, qb, w, wb)


def _tc_layer2_head(aggp, degp, h1, q, qb, w, wb, g_mat, gb,
                    gam1, bet1, gam2, bet2):
  n_nodes, d = h1.shape
  bn = 1000
  grid = n_nodes // bn

  def body(aggp_ref, degp_ref, h1_ref, q_ref, qb_ref, w_ref, wb_ref,
           gm_ref, gb_ref, g1_ref, b1_ref, g2_ref, b2_ref, o_ref, zs_ref):
    i = pl.program_id(0)
    h2 = _pinconv_block(aggp_ref, degp_ref, h1_ref, q_ref, qb_ref,
                        w_ref, wb_ref, d)
    z = jnp.maximum(
        lax.dot_general(h2, gm_ref[...], (((1,), (0,)), ((), ())),
                        preferred_element_type=jnp.float32) + gb_ref[...],
        0.0)
    zs_ref[pl.ds(i * bn, bn), :] = z

    @pl.when(i == grid - 1)
    def _():
      zz = zs_ref[...]
      inv_n = 1.0 / n_nodes
      mu = jnp.sum(zz, axis=0) * inv_n
      c = zz - mu
      var = jnp.sum(c * c, axis=0) * inv_n
      y = g1_ref[...] * c / jnp.sqrt(var + 1e-5) + b1_ref[...]
      mu2 = jnp.sum(y, axis=0) * inv_n
      c2 = y - mu2
      var2 = jnp.sum(c2 * c2, axis=0) * inv_n
      o_ref[...] = g2_ref[...] * c2 / jnp.sqrt(var2 + 1e-5) + b2_ref[...]

  vec = pl.BlockSpec((d,), lambda i: (0,))
  return pl.pallas_call(
      body,
      grid=(grid,),
      in_specs=[
          pl.BlockSpec((NC, bn, d), lambda i: (0, i, 0)),
          pl.BlockSpec((NW, 1, 1, bn), lambda i: (0, i, 0, 0)),
          pl.BlockSpec((bn, d), lambda i: (i, 0)),
          pl.BlockSpec((d, d), lambda i: (0, 0)),
          vec,
          pl.BlockSpec((2 * d, d), lambda i: (0, 0)),
          vec,
          pl.BlockSpec((d, d), lambda i: (0, 0)),
          vec, vec, vec, vec, vec,
      ],
      out_specs=pl.BlockSpec((n_nodes, d), lambda i: (0, 0)),
      out_shape=jax.ShapeDtypeStruct((n_nodes, d), jnp.float32),
      scratch_shapes=[pltpu.VMEM((n_nodes, d), jnp.float32)],
  )(aggp, degp.reshape(NW, grid, 1, bn), h1, q, qb, w, wb, g_mat, gb,
    gam1, bet1, gam2, bet2)


def kernel(h, edge_index, Q0, qb0, W0, wb0, Q1, qb1, W1, wb1, G_mat, Gb, g,
           bn_out_gamma, bn_out_beta, bn_gamma, bn_beta):
  e = edge_index.shape[1]
  d = h.shape[1]
  edges3d = jnp.transpose(edge_index.reshape(2, e // G, G), (1, 0, 2))
  dst1d = edge_index[1]
  zeros_gd = jnp.zeros((640, d), jnp.float32)

  agg1p, degp = _sc_aggregate(h, edges3d, dst1d, zeros_gd, with_deg=True)
  h1 = _tc_layer(agg1p, degp, h, Q0, qb0, W0, wb0)
  (agg2p,) = _sc_aggregate(h1, edges3d, dst1d, zeros_gd, with_deg=False)
  gam1 = g * bn_out_gamma
  bet1 = g * bn_out_beta
  return _tc_layer2_head(agg2p, degp, h1, Q1, qb1, W1, wb1, G_mat, Gb,
                         gam1, bet1, bn_gamma, bn_beta)


# speedup vs baseline: 12.6924x; 1.0272x over previous
import functools

import jax
import jax.numpy as jnp
from jax import lax
from jax.experimental import pallas as pl
from jax.experimental.pallas import tpu as pltpu
from jax.experimental.pallas import tpu_sc as plsc

NC = 2
NS = 16
NW = NC * NS

G = 125
RPL = 8


def _sc_aggregate(h, edges3d, dst1d, zeros_gd, with_deg):
  n_nodes, d = h.shape
  n_rows = edges3d.shape[0] * RPL
  rows_per_tile = n_rows // NW
  n_loads = rows_per_tile // RPL
  n_pairs = n_loads // 2
  epl = RPL * G
  npt = (n_nodes // NS) // 8 * 8
  rem = n_nodes - npt * NS
  assert rows_per_tile % RPL == 0 and n_loads % 2 == 0
  assert rem % 8 == 0

  mesh = plsc.VectorSubcoreMesh(core_axis_name="c", subcore_axis_name="s")

  out_type = [jax.ShapeDtypeStruct((NC, n_nodes, d), jnp.float32)]
  if with_deg:
    out_type.append(jax.ShapeDtypeStruct((NW, 1, n_nodes), jnp.float32))

  scratch = [
      pltpu.VMEM((2 * RPL, G), jnp.int32),
      pltpu.VMEM((2 * RPL, G), jnp.int32),
      pltpu.VMEM((G, d), jnp.float32),
      pltpu.VMEM((G, d), jnp.float32),
      pltpu.VMEM_SHARED((n_nodes, d), jnp.float32),
      pltpu.SemaphoreType.DMA,
      pltpu.SemaphoreType.DMA,
      pltpu.SemaphoreType.DMA,
  ]
  if with_deg:
    scratch += [
        pltpu.VMEM((epl,), jnp.int32),
        pltpu.VMEM((epl,), jnp.int32),
        pltpu.VMEM((n_nodes,), jnp.float32),
    ]

  @functools.partial(
      pl.kernel, mesh=mesh, out_type=out_type, scratch_types=scratch,
      compiler_params=pltpu.CompilerParams(needs_layout_passes=False))
  def agg_kernel(h_hbm, edg_hbm, dstf_hbm, zgd_hbm, *rest):
    if with_deg:
      agg_out, deg_out = rest[0], rest[1]
      (edg_a, edg_b, rows_a, rows_b, acc_sh, sem_a, sem_b, sem_i,
       dstf_a, dstf_b, deg_v) = rest[2:]
    else:
      agg_out = rest[0]
      (edg_a, edg_b, rows_a, rows_b, acc_sh, sem_a, sem_b, sem_i) = rest[1:]
      dstf_a = dstf_b = deg_v = None

    cid = lax.axis_index("c")
    sid = lax.axis_index("s")
    wid = sid * NC + cid
    row0 = wid * rows_per_tile

    bufs = (rows_a, rows_b)
    sems = (sem_a, sem_b)
    edgs = (edg_a, edg_b)
    dstfs = (dstf_a, dstf_b)

    def idx_start(half, load):
      pltpu.async_copy(edg_hbm.at[row0 // RPL + load], edgs[half], sem_i)
      if with_deg:
        pltpu.async_copy(dstf_hbm.at[pl.ds((row0 + load * RPL) * G, epl)],
                         dstfs[half], sem_i)

    def idx_wait(half):
      pltpu.make_async_copy(edg_hbm.at[0], edgs[half], sem_i).wait()
      if with_deg:
        pltpu.make_async_copy(dstf_hbm.at[pl.ds(0, epl)], dstfs[half],
                              sem_i).wait()

    def gather_start(half, j, buf):
      pltpu.async_copy(h_hbm.at[edgs[half].at[j]], bufs[buf], sems[buf])

    def gather_wait(half, j, buf):
      pltpu.make_async_copy(h_hbm.at[edgs[half].at[j]], bufs[buf],
                            sems[buf]).wait()

    ones16 = jnp.ones((16,), jnp.float32)
    tail = epl % 16
    tail_mask = lax.iota(jnp.int32, 16) >= (16 - tail)

    def deg_accumulate(half):
      dstf = dstfs[half]

      def degbody(k, _):
        plsc.addupdate_scatter(deg_v, [dstf[pl.ds(k * 16, 16)]], ones16)
        return 0
      lax.fori_loop(0, epl // 16, degbody, 0, unroll=False)
      if tail:
        plsc.addupdate_scatter(deg_v, [dstf[pl.ds(epl - 16, 16)]], ones16,
                               mask=tail_mask)

    def process(half, cross_issue):
      if with_deg:
        deg_accumulate(half)
      for j in range(RPL):
        if j + 1 < RPL:
          gather_start(half, j + 1, (j + 1) % 2)
        else:
          cross_issue()
        gather_wait(half, j, j % 2)
        pltpu.sync_copy(bufs[j % 2], acc_sh.at[edgs[half].at[RPL + j]],
                        add=True)

    idx_start(0, 0)

    @pl.when(sid < NS - 1)
    def _zero_main():
      pltpu.sync_copy(zgd_hbm.at[pl.ds(0, npt)],
                      acc_sh.at[pl.ds(sid * npt, npt)])

    @pl.when(sid == NS - 1)
    def _zero_tail():
      pltpu.sync_copy(zgd_hbm.at[pl.ds(0, npt + rem)],
                      acc_sh.at[pl.ds(sid * npt, npt + rem)])

    if with_deg:
      def zfill(i, _):
        deg_v[pl.ds(i * 16, 16)] = jnp.zeros((16,), jnp.float32)
        return 0
      lax.fori_loop(0, n_nodes // 16, zfill, 0, unroll=False)

    idx_wait(0)
    gather_start(0, 0, 0)
    plsc.subcore_barrier()

    def body(k, _):
      idx_start(1, 2 * k + 1)

      def cross_ab():
        idx_wait(1)
        gather_start(1, 0, 0)
      process(0, cross_ab)

      @pl.when(k < n_pairs - 1)
      def _prefetch_a():
        idx_start(0, 2 * k + 2)

      def cross_ba():
        @pl.when(k < n_pairs - 1)
        def _():
          idx_wait(0)
          gather_start(0, 0, 0)
      process(1, cross_ba)
      return 0
    lax.fori_loop(0, n_pairs, body, 0, unroll=False)

    plsc.subcore_barrier()

    pltpu.sync_copy(acc_sh.at[pl.ds(sid * npt, npt)],
                    agg_out.at[cid, pl.ds(sid * npt, npt)])

    @pl.when(sid == NS - 1)
    def _dump_tail():
      pltpu.sync_copy(acc_sh.at[pl.ds(NS * npt, rem)],
                      agg_out.at[cid, pl.ds(NS * npt, rem)])

    if with_deg:
      pltpu.sync_copy(deg_v, deg_out.at[wid, 0])

  return agg_kernel(h, edges3d, dst1d, zeros_gd)


def _split_matmul(x, nh, w, wb, d):
  z = (lax.dot_general(x, w[pl.ds(0, d), :], (((1,), (0,)), ((), ())),
                       preferred_element_type=jnp.float32)
       + lax.dot_general(nh, w[pl.ds(d, d), :], (((1,), (0,)), ((), ())),
                         preferred_element_type=jnp.float32)
       + wb[...])
  return jnp.maximum(z, 0.0)


def _pinconv_block(aggp_ref, degp_ref, x_ref, q_ref, qb_ref, w_ref, wb_ref,
                   d):
  agg = aggp_ref[0] + aggp_ref[1]
  deg = jnp.sum(degp_ref[:, 0, 0, :], axis=0)
  deg = jnp.maximum(deg, 1.0)[:, None]
  agg = agg / deg
  nh = jnp.maximum(
      lax.dot_general(agg, q_ref[...], (((1,), (0,)), ((), ())),
                      preferred_element_type=jnp.float32) + qb_ref[...],
      0.0)
  z = _split_matmul(x_ref[...], nh, w_ref, wb_ref, d)
  nrm = jnp.sqrt(jnp.sum(z * z, axis=1, keepdims=True))
  return z / jnp.maximum(nrm, 1e-8)


def _tc_layer(aggp, degp, h, q, qb, w, wb):
  n_nodes, d = h.shape
  bn = 1000
  grid = n_nodes // bn

  def body(aggp_ref, degp_ref, h_ref, q_ref, qb_ref, w_ref, wb_ref, o_ref):
    o_ref[...] = _pinconv_block(aggp_ref, degp_ref, h_ref, q_ref, qb_ref,
                                w_ref, wb_ref, d)

  return pl.pallas_call(
      body,
      grid=(grid,),
      in_specs=[
          pl.BlockSpec((NC, bn, d), lambda i: (0, i, 0)),
          pl.BlockSpec((NW, 1, 1, bn), lambda i: (0, i, 0, 0)),
          pl.BlockSpec((bn, d), lambda i: (i, 0)),
          pl.BlockSpec((d, d), lambda i: (0, 0)),
          pl.BlockSpec((d,), lambda i: (0,)),
          pl.BlockSpec((2 * d, d), lambda i: (0, 0)),
          pl.BlockSpec((d,), lambda i: (0,)),
      ],
      out_specs=pl.BlockSpec((bn, d), lambda i: (i, 0)),
      out_shape=jax.ShapeDtypeStruct((n_nodes, d), jnp.float32),
  )(aggp, degp.reshape(NW, grid, 1, bn), h, q, qb, w, wb)


def _tc_layer2_head(aggp, degp, h1, q, qb, w, wb, g_mat, gb,
                    gam1, bet1, gam2, bet2):
  n_nodes, d = h1.shape
  bn = 1000
  grid = n_nodes // bn

  def body(aggp_ref, degp_ref, h1_ref, q_ref, qb_ref, w_ref, wb_ref,
           gm_ref, gb_ref, g1_ref, b1_ref, g2_ref, b2_ref, o_ref, zs_ref):
    i = pl.program_id(0)
    h2 = _pinconv_block(aggp_ref, degp_ref, h1_ref, q_ref, qb_ref,
                        w_ref, wb_ref, d)
    z = jnp.maximum(
        lax.dot_general(h2, gm_ref[...], (((1,), (0,)), ((), ())),
                        preferred_element_type=jnp.float32) + gb_ref[...],
        0.0)
    zs_ref[pl.ds(i * bn, bn), :] = z

    @pl.when(i == grid - 1)
    def _():
      zz = zs_ref[...]
      inv_n = 1.0 / n_nodes
      mu = jnp.sum(zz, axis=0) * inv_n
      c = zz - mu
      var = jnp.sum(c * c, axis=0) * inv_n
      y = g1_ref[...] * c / jnp.sqrt(var + 1e-5) + b1_ref[...]
      mu2 = jnp.sum(y, axis=0) * inv_n
      c2 = y - mu2
      var2 = jnp.sum(c2 * c2, axis=0) * inv_n
      o_ref[...] = g2_ref[...] * c2 / jnp.sqrt(var2 + 1e-5) + b2_ref[...]

  vec = pl.BlockSpec((d,), lambda i: (0,))
  return pl.pallas_call(
      body,
      grid=(grid,),
      in_specs=[
          pl.BlockSpec((NC, bn, d), lambda i: (0, i, 0)),
          pl.BlockSpec((NW, 1, 1, bn), lambda i: (0, i, 0, 0)),
          pl.BlockSpec((bn, d), lambda i: (i, 0)),
          pl.BlockSpec((d, d), lambda i: (0, 0)),
          vec,
          pl.BlockSpec((2 * d, d), lambda i: (0, 0)),
          vec,
          pl.BlockSpec((d, d), lambda i: (0, 0)),
          vec, vec, vec, vec, vec,
      ],
      out_specs=pl.BlockSpec((n_nodes, d), lambda i: (0, 0)),
      out_shape=jax.ShapeDtypeStruct((n_nodes, d), jnp.float32),
      scratch_shapes=[pltpu.VMEM((n_nodes, d), jnp.float32)],
  )(aggp, degp.reshape(NW, grid, 1, bn), h1, q, qb, w, wb, g_mat, gb,
    gam1, bet1, gam2, bet2)


def kernel(h, edge_index, Q0, qb0, W0, wb0, Q1, qb1, W1, wb1, G_mat, Gb, g,
           bn_out_gamma, bn_out_beta, bn_gamma, bn_beta):
  e = edge_index.shape[1]
  d = h.shape[1]
  edges3d = jnp.transpose(
      edge_index.reshape(2, e // (G * RPL), RPL, G),
      (1, 0, 2, 3)).reshape(e // (G * RPL), 2 * RPL, G)
  dst1d = edge_index[1]
  zeros_gd = jnp.zeros((640, d), jnp.float32)

  agg1p, degp = _sc_aggregate(h, edges3d, dst1d, zeros_gd, with_deg=True)
  h1 = _tc_layer(agg1p, degp, h, Q0, qb0, W0, wb0)
  (agg2p,) = _sc_aggregate(h1, edges3d, dst1d, zeros_gd, with_deg=False)
  gam1 = g * bn_out_gamma
  bet1 = g * bn_out_beta
  return _tc_layer2_head(agg2p, degp, h1, Q1, qb1, W1, wb1, G_mat, Gb,
                         gam1, bet1, bn_gamma, bn_beta)


# speedup vs baseline: 13.1750x; 1.0380x over previous
import functools

import jax
import jax.numpy as jnp
from jax import lax
from jax.experimental import pallas as pl
from jax.experimental.pallas import tpu as pltpu
from jax.experimental.pallas import tpu_sc as plsc

NC = 2
NS = 16
NW = NC * NS

G = 125
RPL = 8


def _sc_aggregate(h, edges3d, eflat, zeros_gd, with_deg):
  n_nodes, d = h.shape
  n_edges = eflat.shape[0] // 2
  n_rows = edges3d.shape[1]
  rows_per_tile = n_rows // NW
  n_loads = rows_per_tile // RPL
  n_pairs = n_loads // 2
  epl = RPL * G
  npt = (n_nodes // NS) // 8 * 8
  rem = n_nodes - npt * NS
  assert rows_per_tile % RPL == 0 and n_loads % 2 == 0
  assert rem % 8 == 0

  mesh = plsc.VectorSubcoreMesh(core_axis_name="c", subcore_axis_name="s")

  out_type = [jax.ShapeDtypeStruct((NC, n_nodes, d), jnp.float32)]
  if with_deg:
    out_type.append(jax.ShapeDtypeStruct((NW, 1, n_nodes), jnp.float32))

  scratch = [
      pltpu.VMEM((RPL, G), jnp.int32),
      pltpu.VMEM((RPL, G), jnp.int32),
      pltpu.VMEM((RPL, G), jnp.int32),
      pltpu.VMEM((RPL, G), jnp.int32),
      pltpu.VMEM((G, d), jnp.float32),
      pltpu.VMEM((G, d), jnp.float32),
      pltpu.VMEM_SHARED((n_nodes, d), jnp.float32),
      pltpu.SemaphoreType.DMA,
      pltpu.SemaphoreType.DMA,
      pltpu.SemaphoreType.DMA,
  ]
  if with_deg:
    scratch += [
        pltpu.VMEM((epl,), jnp.int32),
        pltpu.VMEM((epl,), jnp.int32),
        pltpu.VMEM((n_nodes,), jnp.float32),
    ]

  @functools.partial(
      pl.kernel, mesh=mesh, out_type=out_type, scratch_types=scratch,
      compiler_params=pltpu.CompilerParams(needs_layout_passes=False))
  def agg_kernel(h_hbm, edg_hbm, dstf_hbm, zgd_hbm, *rest):
    if with_deg:
      agg_out, deg_out = rest[0], rest[1]
      (src_a, src_b, dst_a, dst_b, rows_a, rows_b, acc_sh,
       sem_a, sem_b, sem_i, dstf_a, dstf_b, deg_v) = rest[2:]
    else:
      agg_out = rest[0]
      (src_a, src_b, dst_a, dst_b, rows_a, rows_b, acc_sh,
       sem_a, sem_b, sem_i) = rest[1:]
      dstf_a = dstf_b = deg_v = None

    cid = lax.axis_index("c")
    sid = lax.axis_index("s")
    wid = sid * NC + cid
    row0 = wid * rows_per_tile

    bufs = (rows_a, rows_b)
    sems = (sem_a, sem_b)
    srcs = (src_a, src_b)
    dsts = (dst_a, dst_b)
    dstfs = (dstf_a, dstf_b)

    def idx_start(half, load):
      pltpu.async_copy(edg_hbm.at[0, pl.ds(row0 + load * RPL, RPL)],
                       srcs[half], sem_i)
      pltpu.async_copy(edg_hbm.at[1, pl.ds(row0 + load * RPL, RPL)],
                       dsts[half], sem_i)
      if with_deg:
        pltpu.async_copy(
            dstf_hbm.at[pl.ds(n_edges + (row0 + load * RPL) * G, epl)],
            dstfs[half], sem_i)

    def idx_wait(half):
      pltpu.make_async_copy(edg_hbm.at[0, pl.ds(0, RPL)], srcs[half],
                            sem_i).wait()
      pltpu.make_async_copy(edg_hbm.at[1, pl.ds(0, RPL)], dsts[half],
                            sem_i).wait()
      if with_deg:
        pltpu.make_async_copy(dstf_hbm.at[pl.ds(0, epl)], dstfs[half],
                              sem_i).wait()

    def gather_start(half, j, buf):
      pltpu.async_copy(h_hbm.at[srcs[half].at[j]], bufs[buf], sems[buf])

    def gather_wait(half, j, buf):
      pltpu.make_async_copy(h_hbm.at[srcs[half].at[j]], bufs[buf],
                            sems[buf]).wait()

    ones16 = jnp.ones((16,), jnp.float32)
    tail = epl % 16
    tail_mask = lax.iota(jnp.int32, 16) >= (16 - tail)

    def deg_accumulate(half):
      dstf = dstfs[half]

      def degbody(k, _):
        plsc.addupdate_scatter(deg_v, [dstf[pl.ds(k * 16, 16)]], ones16)
        return 0
      lax.fori_loop(0, epl // 16, degbody, 0, unroll=False)
      if tail:
        plsc.addupdate_scatter(deg_v, [dstf[pl.ds(epl - 16, 16)]], ones16,
                               mask=tail_mask)

    def process(half, cross_issue):
      if with_deg:
        deg_accumulate(half)
      for j in range(RPL):
        if j + 1 < RPL:
          gather_start(half, j + 1, (j + 1) % 2)
        else:
          cross_issue()
        gather_wait(half, j, j % 2)
        pltpu.sync_copy(bufs[j % 2], acc_sh.at[dsts[half].at[j]],
                        add=True)

    idx_start(0, 0)

    @pl.when(sid < NS - 1)
    def _zero_main():
      pltpu.sync_copy(zgd_hbm.at[pl.ds(0, npt)],
                      acc_sh.at[pl.ds(sid * npt, npt)])

    @pl.when(sid == NS - 1)
    def _zero_tail():
      pltpu.sync_copy(zgd_hbm.at[pl.ds(0, npt + rem)],
                      acc_sh.at[pl.ds(sid * npt, npt + rem)])

    if with_deg:
      def zfill(i, _):
        deg_v[pl.ds(i * 16, 16)] = jnp.zeros((16,), jnp.float32)
        return 0
      lax.fori_loop(0, n_nodes // 16, zfill, 0, unroll=False)

    idx_wait(0)
    gather_start(0, 0, 0)
    plsc.subcore_barrier()

    def body(k, _):
      idx_start(1, 2 * k + 1)

      def cross_ab():
        idx_wait(1)
        gather_start(1, 0, 0)
      process(0, cross_ab)

      @pl.when(k < n_pairs - 1)
      def _prefetch_a():
        idx_start(0, 2 * k + 2)

      def cross_ba():
        @pl.when(k < n_pairs - 1)
        def _():
          idx_wait(0)
          gather_start(0, 0, 0)
      process(1, cross_ba)
      return 0
    lax.fori_loop(0, n_pairs, body, 0, unroll=False)

    plsc.subcore_barrier()

    pltpu.sync_copy(acc_sh.at[pl.ds(sid * npt, npt)],
                    agg_out.at[cid, pl.ds(sid * npt, npt)])

    @pl.when(sid == NS - 1)
    def _dump_tail():
      pltpu.sync_copy(acc_sh.at[pl.ds(NS * npt, rem)],
                      agg_out.at[cid, pl.ds(NS * npt, rem)])

    if with_deg:
      pltpu.sync_copy(deg_v, deg_out.at[wid, 0])

  return agg_kernel(h, edges3d, eflat, zeros_gd)


def _split_matmul(x, nh, w, wb, d):
  z = (lax.dot_general(x, w[pl.ds(0, d), :], (((1,), (0,)), ((), ())),
                       preferred_element_type=jnp.float32)
       + lax.dot_general(nh, w[pl.ds(d, d), :], (((1,), (0,)), ((), ())),
                         preferred_element_type=jnp.float32)
       + wb[...])
  return jnp.maximum(z, 0.0)


def _pinconv_block(aggp_ref, degp_ref, x_ref, q_ref, qb_ref, w_ref, wb_ref,
                   d):
  agg = aggp_ref[0] + aggp_ref[1]
  deg = jnp.sum(degp_ref[:, 0, 0, :], axis=0)
  deg = jnp.maximum(deg, 1.0)[:, None]
  agg = agg / deg
  nh = jnp.maximum(
      lax.dot_general(agg, q_ref[...], (((1,), (0,)), ((), ())),
                      preferred_element_type=jnp.float32) + qb_ref[...],
      0.0)
  z = _split_matmul(x_ref[...], nh, w_ref, wb_ref, d)
  nrm = jnp.sqrt(jnp.sum(z * z, axis=1, keepdims=True))
  return z / jnp.maximum(nrm, 1e-8)


def _tc_layer(aggp, degp, h, q, qb, w, wb):
  n_nodes, d = h.shape
  bn = 1000
  grid = n_nodes // bn

  def body(aggp_ref, degp_ref, h_ref, q_ref, qb_ref, w_ref, wb_ref, o_ref):
    o_ref[...] = _pinconv_block(aggp_ref, degp_ref, h_ref, q_ref, qb_ref,
                                w_ref, wb_ref, d)

  return pl.pallas_call(
      body,
      grid=(grid,),
      in_specs=[
          pl.BlockSpec((NC, bn, d), lambda i: (0, i, 0)),
          pl.BlockSpec((NW, 1, 1, bn), lambda i: (0, i, 0, 0)),
          pl.BlockSpec((bn, d), lambda i: (i, 0)),
          pl.BlockSpec((d, d), lambda i: (0, 0)),
          pl.BlockSpec((d,), lambda i: (0,)),
          pl.BlockSpec((2 * d, d), lambda i: (0, 0)),
          pl.BlockSpec((d,), lambda i: (0,)),
      ],
      out_specs=pl.BlockSpec((bn, d), lambda i: (i, 0)),
      out_shape=jax.ShapeDtypeStruct((n_nodes, d), jnp.float32),
  )(aggp, degp.reshape(NW, grid, 1, bn), h, q, qb, w, wb)


def _tc_layer2_head(aggp, degp, h1, q, qb, w, wb, g_mat, gb,
                    gam1, bet1, gam2, bet2):
  n_nodes, d = h1.shape
  bn = 1000
  grid = n_nodes // bn

  def body(aggp_ref, degp_ref, h1_ref, q_ref, qb_ref, w_ref, wb_ref,
           gm_ref, gb_ref, g1_ref, b1_ref, g2_ref, b2_ref, o_ref, zs_ref):
    i = pl.program_id(0)
    h2 = _pinconv_block(aggp_ref, degp_ref, h1_ref, q_ref, qb_ref,
                        w_ref, wb_ref, d)
    z = jnp.maximum(
        lax.dot_general(h2, gm_ref[...], (((1,), (0,)), ((), ())),
                        preferred_element_type=jnp.float32) + gb_ref[...],
        0.0)
    zs_ref[pl.ds(i * bn, bn), :] = z

    @pl.when(i == grid - 1)
    def _():
      zz = zs_ref[...]
      inv_n = 1.0 / n_nodes
      mu = jnp.sum(zz, axis=0) * inv_n
      c = zz - mu
      var = jnp.sum(c * c, axis=0) * inv_n
      y = g1_ref[...] * c / jnp.sqrt(var + 1e-5) + b1_ref[...]
      mu2 = jnp.sum(y, axis=0) * inv_n
      c2 = y - mu2
      var2 = jnp.sum(c2 * c2, axis=0) * inv_n
      o_ref[...] = g2_ref[...] * c2 / jnp.sqrt(var2 + 1e-5) + b2_ref[...]

  vec = pl.BlockSpec((d,), lambda i: (0,))
  return pl.pallas_call(
      body,
      grid=(grid,),
      in_specs=[
          pl.BlockSpec((NC, bn, d), lambda i: (0, i, 0)),
          pl.BlockSpec((NW, 1, 1, bn), lambda i: (0, i, 0, 0)),
          pl.BlockSpec((bn, d), lambda i: (i, 0)),
          pl.BlockSpec((d, d), lambda i: (0, 0)),
          vec,
          pl.BlockSpec((2 * d, d), lambda i: (0, 0)),
          vec,
          pl.BlockSpec((d, d), lambda i: (0, 0)),
          vec, vec, vec, vec, vec,
      ],
      out_specs=pl.BlockSpec((n_nodes, d), lambda i: (0, 0)),
      out_shape=jax.ShapeDtypeStruct((n_nodes, d), jnp.float32),
      scratch_shapes=[pltpu.VMEM((n_nodes, d), jnp.float32)],
  )(aggp, degp.reshape(NW, grid, 1, bn), h1, q, qb, w, wb, g_mat, gb,
    gam1, bet1, gam2, bet2)


def kernel(h, edge_index, Q0, qb0, W0, wb0, Q1, qb1, W1, wb1, G_mat, Gb, g,
           bn_out_gamma, bn_out_beta, bn_gamma, bn_beta):
  e = edge_index.shape[1]
  d = h.shape[1]
  edges3d = edge_index.reshape(2, e // G, G)
  eflat = edge_index.reshape(2 * e)
  zeros_gd = jnp.zeros((640, d), jnp.float32)

  agg1p, degp = _sc_aggregate(h, edges3d, eflat, zeros_gd, with_deg=True)
  h1 = _tc_layer(agg1p, degp, h, Q0, qb0, W0, wb0)
  (agg2p,) = _sc_aggregate(h1, edges3d, eflat, zeros_gd, with_deg=False)
  gam1 = g * bn_out_gamma
  bet1 = g * bn_out_beta
  return _tc_layer2_head(agg2p, degp, h1, Q1, qb1, W1, wb1, G_mat, Gb,
                         gam1, bet1, bn_gamma, bn_beta)


# speedup vs baseline: 13.2696x; 1.0072x over previous
import functools

import jax
import jax.numpy as jnp
from jax import lax
from jax.experimental import pallas as pl
from jax.experimental.pallas import tpu as pltpu
from jax.experimental.pallas import tpu_sc as plsc

NC = 2
NS = 16
NW = NC * NS

G = 125
RPL = 8


def _sc_aggregate(h, edges3d, eflat, zeros_gd, with_deg):
  n_nodes, d = h.shape
  n_edges = eflat.shape[0] // 2
  n_rows = edges3d.shape[1]
  rows_per_tile = n_rows // NW
  n_loads = rows_per_tile // RPL
  n_pairs = n_loads // 2
  epl = RPL * G
  npt = (n_nodes // NS) // 8 * 8
  rem = n_nodes - npt * NS
  assert rows_per_tile % RPL == 0 and n_loads % 2 == 0
  assert rem % 8 == 0

  mesh = plsc.VectorSubcoreMesh(core_axis_name="c", subcore_axis_name="s")

  out_type = [jax.ShapeDtypeStruct((NC, n_nodes, d), jnp.float32)]
  if with_deg:
    out_type.append(jax.ShapeDtypeStruct((NW, 1, n_nodes), jnp.float32))

  scratch = [
      pltpu.VMEM((RPL, G), jnp.int32),
      pltpu.VMEM((RPL, G), jnp.int32),
      pltpu.VMEM((RPL, G), jnp.int32),
      pltpu.VMEM((RPL, G), jnp.int32),
      pltpu.VMEM((G, d), jnp.float32),
      pltpu.VMEM((G, d), jnp.float32),
      pltpu.VMEM_SHARED((n_nodes, d), jnp.float32),
      pltpu.SemaphoreType.DMA,
      pltpu.SemaphoreType.DMA,
      pltpu.SemaphoreType.DMA,
  ]
  if with_deg:
    scratch += [
        pltpu.VMEM((epl,), jnp.int32),
        pltpu.VMEM((epl,), jnp.int32),
        pltpu.VMEM((n_nodes,), jnp.float32),
    ]

  @functools.partial(
      pl.kernel, mesh=mesh, out_type=out_type, scratch_types=scratch,
      compiler_params=pltpu.CompilerParams(needs_layout_passes=False))
  def agg_kernel(h_hbm, edg_hbm, dstf_hbm, zgd_hbm, *rest):
    if with_deg:
      agg_out, deg_out = rest[0], rest[1]
      (src_a, src_b, dst_a, dst_b, rows_a, rows_b, acc_sh,
       sem_a, sem_b, sem_i, dstf_a, dstf_b, deg_v) = rest[2:]
    else:
      agg_out = rest[0]
      (src_a, src_b, dst_a, dst_b, rows_a, rows_b, acc_sh,
       sem_a, sem_b, sem_i) = rest[1:]
      dstf_a = dstf_b = deg_v = None

    cid = lax.axis_index("c")
    sid = lax.axis_index("s")
    wid = sid * NC + cid
    row0 = wid * rows_per_tile

    bufs = (rows_a, rows_b)
    sems = (sem_a, sem_b)
    srcs = (src_a, src_b)
    dsts = (dst_a, dst_b)
    dstfs = (dstf_a, dstf_b)

    def idx_start(half, load):
      pltpu.async_copy(edg_hbm.at[0, pl.ds(row0 + load * RPL, RPL)],
                       srcs[half], sem_i)
      pltpu.async_copy(edg_hbm.at[1, pl.ds(row0 + load * RPL, RPL)],
                       dsts[half], sem_i)
      if with_deg:
        pltpu.async_copy(
            dstf_hbm.at[pl.ds(n_edges + (row0 + load * RPL) * G, epl)],
            dstfs[half], sem_i)

    def idx_wait(half):
      pltpu.make_async_copy(edg_hbm.at[0, pl.ds(0, RPL)], srcs[half],
                            sem_i).wait()
      pltpu.make_async_copy(edg_hbm.at[1, pl.ds(0, RPL)], dsts[half],
                            sem_i).wait()
      if with_deg:
        pltpu.make_async_copy(dstf_hbm.at[pl.ds(0, epl)], dstfs[half],
                              sem_i).wait()

    def gather_start(half, j, buf):
      pltpu.async_copy(h_hbm.at[srcs[half].at[j]], bufs[buf], sems[buf])

    def gather_wait(half, j, buf):
      pltpu.make_async_copy(h_hbm.at[srcs[half].at[j]], bufs[buf],
                            sems[buf]).wait()

    ones16 = jnp.ones((16,), jnp.float32)
    tail = epl % 16
    tail_mask = lax.iota(jnp.int32, 16) >= (16 - tail)

    def deg_accumulate(half):
      dstf = dstfs[half]

      def degbody(k, _):
        plsc.addupdate_scatter(deg_v, [dstf[pl.ds(k * 16, 16)]], ones16)
        return 0
      lax.fori_loop(0, epl // 16, degbody, 0, unroll=False)
      if tail:
        plsc.addupdate_scatter(deg_v, [dstf[pl.ds(epl - 16, 16)]], ones16,
                               mask=tail_mask)

    def process(half, cross_issue):
      if with_deg:
        deg_accumulate(half)
      for j in range(RPL):
        if j + 1 < RPL:
          gather_start(half, j + 1, (j + 1) % 2)
        else:
          cross_issue()
        gather_wait(half, j, j % 2)
        pltpu.sync_copy(bufs[j % 2], acc_sh.at[dsts[half].at[j]],
                        add=True)

    idx_start(0, 0)

    @pl.when(sid < NS - 1)
    def _zero_main():
      pltpu.sync_copy(zgd_hbm.at[pl.ds(0, npt)],
                      acc_sh.at[pl.ds(sid * npt, npt)])

    @pl.when(sid == NS - 1)
    def _zero_tail():
      pltpu.sync_copy(zgd_hbm.at[pl.ds(0, npt + rem)],
                      acc_sh.at[pl.ds(sid * npt, npt + rem)])

    if with_deg:
      def zfill(i, _):
        deg_v[pl.ds(i * 16, 16)] = jnp.zeros((16,), jnp.float32)
        return 0
      lax.fori_loop(0, n_nodes // 16, zfill, 0, unroll=False)

    idx_wait(0)
    gather_start(0, 0, 0)
    plsc.subcore_barrier()

    def body(k, _):
      idx_start(1, 2 * k + 1)

      def cross_ab():
        idx_wait(1)
        gather_start(1, 0, 0)
      process(0, cross_ab)

      @pl.when(k < n_pairs - 1)
      def _prefetch_a():
        idx_start(0, 2 * k + 2)

      def cross_ba():
        @pl.when(k < n_pairs - 1)
        def _():
          idx_wait(0)
          gather_start(0, 0, 0)
      process(1, cross_ba)
      return 0
    lax.fori_loop(0, n_pairs, body, 0, unroll=False)

    plsc.subcore_barrier()

    pltpu.sync_copy(acc_sh.at[pl.ds(sid * npt, npt)],
                    agg_out.at[cid, pl.ds(sid * npt, npt)])

    @pl.when(sid == NS - 1)
    def _dump_tail():
      pltpu.sync_copy(acc_sh.at[pl.ds(NS * npt, rem)],
                      agg_out.at[cid, pl.ds(NS * npt, rem)])

    if with_deg:
      pltpu.sync_copy(deg_v, deg_out.at[wid, 0])

  return agg_kernel(h, edges3d, eflat, zeros_gd)


def _split_matmul(x, nh, w, wb, d):
  z = (lax.dot_general(x, w[pl.ds(0, d), :], (((1,), (0,)), ((), ())),
                       preferred_element_type=jnp.float32)
       + lax.dot_general(nh, w[pl.ds(d, d), :], (((1,), (0,)), ((), ())),
                         preferred_element_type=jnp.float32)
       + wb[...])
  return jnp.maximum(z, 0.0)


def _pinconv_block(aggp_ref, degp_ref, x_ref, q_ref, qb_ref, w_ref, wb_ref,
                   d):
  agg = aggp_ref[0] + aggp_ref[1]
  deg = jnp.sum(degp_ref[:, 0, 0, :], axis=0)
  deg = jnp.maximum(deg, 1.0)[:, None]
  agg = agg / deg
  nh = jnp.maximum(
      lax.dot_general(agg, q_ref[...], (((1,), (0,)), ((), ())),
                      preferred_element_type=jnp.float32) + qb_ref[...],
      0.0)
  z = _split_matmul(x_ref[...], nh, w_ref, wb_ref, d)
  nrm = jnp.sqrt(jnp.sum(z * z, axis=1, keepdims=True))
  return z / jnp.maximum(nrm, 1e-8)


def _tc_layer(aggp, degp, h, q, qb, w, wb):
  n_nodes, d = h.shape
  bn = 1000
  grid = n_nodes // bn

  def body(aggp_ref, degp_ref, h_ref, q_ref, qb_ref, w_ref, wb_ref, o_ref):
    o_ref[...] = _pinconv_block(aggp_ref, degp_ref, h_ref, q_ref, qb_ref,
                                w_ref, wb_ref, d)

  return pl.pallas_call(
      body,
      grid=(grid,),
      in_specs=[
          pl.BlockSpec((NC, bn, d), lambda i: (0, i, 0)),
          pl.BlockSpec((NW, 1, 1, bn), lambda i: (0, i, 0, 0)),
          pl.BlockSpec((bn, d), lambda i: (i, 0)),
          pl.BlockSpec((d, d), lambda i: (0, 0)),
          pl.BlockSpec((d,), lambda i: (0,)),
          pl.BlockSpec((2 * d, d), lambda i: (0, 0)),
          pl.BlockSpec((d,), lambda i: (0,)),
      ],
      out_specs=pl.BlockSpec((bn, d), lambda i: (i, 0)),
      out_shape=jax.ShapeDtypeStruct((n_nodes, d), jnp.float32),
  )(aggp, degp.reshape(NW, grid, 1, bn), h, q, qb, w, wb)


def _tc_layer2_head(aggp, degp, h1, q, qb, w, wb, g_mat, gb,
                    gam1, bet1, gam2, bet2):
  n_nodes, d = h1.shape
  bn = 1000
  grid = n_nodes // bn

  def body(aggp_ref, degp_ref, h1_ref, q_ref, qb_ref, w_ref, wb_ref,
           gm_ref, gb_ref, g1_ref, b1_ref, g2_ref, b2_ref, o_ref, zs_ref):
    i = pl.program_id(0)
    h2 = _pinconv_block(aggp_ref, degp_ref, h1_ref, q_ref, qb_ref,
                        w_ref, wb_ref, d)
    z = jnp.maximum(
        lax.dot_general(h2, gm_ref[...], (((1,), (0,)), ((), ())),
                        preferred_element_type=jnp.float32) + gb_ref[...],
        0.0)
    zs_ref[pl.ds(i * bn, bn), :] = z

    @pl.when(i == grid - 1)
    def _():
      zz = zs_ref[...]
      inv_n = 1.0 / n_nodes
      mu = jnp.sum(zz, axis=0) * inv_n
      c = zz - mu
      var = jnp.sum(c * c, axis=0) * inv_n
      a1 = g1_ref[...] / jnp.sqrt(var + 1e-5)
      a2 = g2_ref[...] / jnp.sqrt(a1 * a1 * var + 1e-5)
      o_ref[...] = (a2 * a1) * c + b2_ref[...]

  vec = pl.BlockSpec((d,), lambda i: (0,))
  return pl.pallas_call(
      body,
      grid=(grid,),
      in_specs=[
          pl.BlockSpec((NC, bn, d), lambda i: (0, i, 0)),
          pl.BlockSpec((NW, 1, 1, bn), lambda i: (0, i, 0, 0)),
          pl.BlockSpec((bn, d), lambda i: (i, 0)),
          pl.BlockSpec((d, d), lambda i: (0, 0)),
          vec,
          pl.BlockSpec((2 * d, d), lambda i: (0, 0)),
          vec,
          pl.BlockSpec((d, d), lambda i: (0, 0)),
          vec, vec, vec, vec, vec,
      ],
      out_specs=pl.BlockSpec((n_nodes, d), lambda i: (0, 0)),
      out_shape=jax.ShapeDtypeStruct((n_nodes, d), jnp.float32),
      scratch_shapes=[pltpu.VMEM((n_nodes, d), jnp.float32)],
  )(aggp, degp.reshape(NW, grid, 1, bn), h1, q, qb, w, wb, g_mat, gb,
    gam1, bet1, gam2, bet2)


def kernel(h, edge_index, Q0, qb0, W0, wb0, Q1, qb1, W1, wb1, G_mat, Gb, g,
           bn_out_gamma, bn_out_beta, bn_gamma, bn_beta):
  e = edge_index.shape[1]
  d = h.shape[1]
  edges3d = edge_index.reshape(2, e // G, G)
  eflat = edge_index.reshape(2 * e)
  zeros_gd = jnp.zeros((640, d), jnp.float32)

  agg1p, degp = _sc_aggregate(h, edges3d, eflat, zeros_gd, with_deg=True)
  h1 = _tc_layer(agg1p, degp, h, Q0, qb0, W0, wb0)
  (agg2p,) = _sc_aggregate(h1, edges3d, eflat, zeros_gd, with_deg=False)
  gam1 = g * bn_out_gamma
  bet1 = g * bn_out_beta
  return _tc_layer2_head(agg2p, degp, h1, Q1, qb1, W1, wb1, G_mat, Gb,
                         gam1, bet1, bn_gamma, bn_beta)


# speedup vs baseline: 13.3130x; 1.0033x over previous
import functools

import jax
import jax.numpy as jnp
from jax import lax
from jax.experimental import pallas as pl
from jax.experimental.pallas import tpu as pltpu
from jax.experimental.pallas import tpu_sc as plsc

NC = 2
NS = 16
NW = NC * NS

G = 125
RPL = 8


def _sc_aggregate(h, edges3d, eflat, zeros_gd, with_deg):
  n_nodes, d = h.shape
  n_edges = eflat.shape[0] // 2
  n_rows = edges3d.shape[1]
  rows_per_tile = n_rows // NW
  n_loads = rows_per_tile // RPL
  n_pairs = n_loads // 2
  epl = RPL * G
  npt = (n_nodes // NS) // 8 * 8
  rem = n_nodes - npt * NS
  assert rows_per_tile % RPL == 0 and n_loads % 2 == 0
  assert rem % 8 == 0

  mesh = plsc.VectorSubcoreMesh(core_axis_name="c", subcore_axis_name="s")

  out_type = [jax.ShapeDtypeStruct((NC, n_nodes, d), jnp.float32)]
  if with_deg:
    out_type.append(jax.ShapeDtypeStruct((NW, 1, n_nodes), jnp.float32))

  scratch = [
      pltpu.VMEM((RPL, G), jnp.int32),
      pltpu.VMEM((RPL, G), jnp.int32),
      pltpu.VMEM((RPL, G), jnp.int32),
      pltpu.VMEM((RPL, G), jnp.int32),
      pltpu.VMEM((G, d), jnp.float32),
      pltpu.VMEM((G, d), jnp.float32),
      pltpu.VMEM_SHARED((n_nodes, d), jnp.float32),
      pltpu.SemaphoreType.DMA,
      pltpu.SemaphoreType.DMA,
      pltpu.SemaphoreType.DMA,
  ]
  if with_deg:
    scratch += [
        pltpu.VMEM((epl,), jnp.int32),
        pltpu.VMEM((epl,), jnp.int32),
        pltpu.VMEM((n_nodes,), jnp.float32),
    ]

  @functools.partial(
      pl.kernel, mesh=mesh, out_type=out_type, scratch_types=scratch,
      compiler_params=pltpu.CompilerParams(needs_layout_passes=False))
  def agg_kernel(h_hbm, edg_hbm, dstf_hbm, zgd_hbm, *rest):
    if with_deg:
      agg_out, deg_out = rest[0], rest[1]
      (src_a, src_b, dst_a, dst_b, rows_a, rows_b, acc_sh,
       sem_a, sem_b, sem_i, dstf_a, dstf_b, deg_v) = rest[2:]
    else:
      agg_out = rest[0]
      (src_a, src_b, dst_a, dst_b, rows_a, rows_b, acc_sh,
       sem_a, sem_b, sem_i) = rest[1:]
      dstf_a = dstf_b = deg_v = None

    cid = lax.axis_index("c")
    sid = lax.axis_index("s")
    wid = sid * NC + cid
    row0 = wid * rows_per_tile

    bufs = (rows_a, rows_b)
    sems = (sem_a, sem_b)
    srcs = (src_a, src_b)
    dsts = (dst_a, dst_b)
    dstfs = (dstf_a, dstf_b)

    def idx_start(half, load):
      pltpu.async_copy(edg_hbm.at[0, pl.ds(row0 + load * RPL, RPL)],
                       srcs[half], sem_i)
      pltpu.async_copy(edg_hbm.at[1, pl.ds(row0 + load * RPL, RPL)],
                       dsts[half], sem_i)
      if with_deg:
        pltpu.async_copy(
            dstf_hbm.at[pl.ds(n_edges + (row0 + load * RPL) * G, epl)],
            dstfs[half], sem_i)

    def idx_wait(half):
      pltpu.make_async_copy(edg_hbm.at[0, pl.ds(0, RPL)], srcs[half],
                            sem_i).wait()
      pltpu.make_async_copy(edg_hbm.at[1, pl.ds(0, RPL)], dsts[half],
                            sem_i).wait()
      if with_deg:
        pltpu.make_async_copy(dstf_hbm.at[pl.ds(0, epl)], dstfs[half],
                              sem_i).wait()

    def gather_start(half, j, buf):
      pltpu.async_copy(h_hbm.at[srcs[half].at[j]], bufs[buf], sems[buf])

    def gather_wait(half, j, buf):
      pltpu.make_async_copy(h_hbm.at[srcs[half].at[j]], bufs[buf],
                            sems[buf]).wait()

    ones16 = jnp.ones((16,), jnp.float32)
    tail = epl % 16
    tail_mask = lax.iota(jnp.int32, 16) >= (16 - tail)
    n_degv = epl // 16
    degv_per_j = -(-n_degv // RPL)

    def deg_chunk(half, j):
      dstf = dstfs[half]
      for k in range(j * degv_per_j, min((j + 1) * degv_per_j, n_degv)):
        plsc.addupdate_scatter(deg_v, [dstf[pl.ds(k * 16, 16)]], ones16)
      if tail and j == RPL - 1:
        plsc.addupdate_scatter(deg_v, [dstf[pl.ds(epl - 16, 16)]], ones16,
                               mask=tail_mask)

    def process(half, cross_issue):
      for j in range(RPL):
        if j + 1 < RPL:
          gather_start(half, j + 1, (j + 1) % 2)
        else:
          cross_issue()
        if with_deg:
          deg_chunk(half, j)
        gather_wait(half, j, j % 2)
        pltpu.sync_copy(bufs[j % 2], acc_sh.at[dsts[half].at[j]],
                        add=True)

    idx_start(0, 0)

    @pl.when(sid < NS - 1)
    def _zero_main():
      pltpu.sync_copy(zgd_hbm.at[pl.ds(0, npt)],
                      acc_sh.at[pl.ds(sid * npt, npt)])

    @pl.when(sid == NS - 1)
    def _zero_tail():
      pltpu.sync_copy(zgd_hbm.at[pl.ds(0, npt + rem)],
                      acc_sh.at[pl.ds(sid * npt, npt + rem)])

    if with_deg:
      def zfill(i, _):
        deg_v[pl.ds(i * 16, 16)] = jnp.zeros((16,), jnp.float32)
        return 0
      lax.fori_loop(0, n_nodes // 16, zfill, 0, unroll=False)

    idx_wait(0)
    gather_start(0, 0, 0)
    plsc.subcore_barrier()

    def body(k, _):
      idx_start(1, 2 * k + 1)

      def cross_ab():
        idx_wait(1)
        gather_start(1, 0, 0)
      process(0, cross_ab)

      @pl.when(k < n_pairs - 1)
      def _prefetch_a():
        idx_start(0, 2 * k + 2)

      def cross_ba():
        @pl.when(k < n_pairs - 1)
        def _():
          idx_wait(0)
          gather_start(0, 0, 0)
      process(1, cross_ba)
      return 0
    lax.fori_loop(0, n_pairs, body, 0, unroll=False)

    plsc.subcore_barrier()

    pltpu.sync_copy(acc_sh.at[pl.ds(sid * npt, npt)],
                    agg_out.at[cid, pl.ds(sid * npt, npt)])

    @pl.when(sid == NS - 1)
    def _dump_tail():
      pltpu.sync_copy(acc_sh.at[pl.ds(NS * npt, rem)],
                      agg_out.at[cid, pl.ds(NS * npt, rem)])

    if with_deg:
      pltpu.sync_copy(deg_v, deg_out.at[wid, 0])

  return agg_kernel(h, edges3d, eflat, zeros_gd)


def _split_matmul(x, nh, w, wb, d):
  z = (lax.dot_general(x, w[pl.ds(0, d), :], (((1,), (0,)), ((), ())),
                       preferred_element_type=jnp.float32)
       + lax.dot_general(nh, w[pl.ds(d, d), :], (((1,), (0,)), ((), ())),
                         preferred_element_type=jnp.float32)
       + wb[...])
  return jnp.maximum(z, 0.0)


def _pinconv_block(aggp_ref, degp_ref, x_ref, q_ref, qb_ref, w_ref, wb_ref,
                   d):
  agg = aggp_ref[0] + aggp_ref[1]
  deg = jnp.sum(degp_ref[:, 0, 0, :], axis=0)
  deg = jnp.maximum(deg, 1.0)[:, None]
  agg = agg / deg
  nh = jnp.maximum(
      lax.dot_general(agg, q_ref[...], (((1,), (0,)), ((), ())),
                      preferred_element_type=jnp.float32) + qb_ref[...],
      0.0)
  z = _split_matmul(x_ref[...], nh, w_ref, wb_ref, d)
  nrm = jnp.sqrt(jnp.sum(z * z, axis=1, keepdims=True))
  return z / jnp.maximum(nrm, 1e-8)


def _tc_layer(aggp, degp, h, q, qb, w, wb):
  n_nodes, d = h.shape
  bn = 1000
  grid = n_nodes // bn

  def body(aggp_ref, degp_ref, h_ref, q_ref, qb_ref, w_ref, wb_ref, o_ref):
    o_ref[...] = _pinconv_block(aggp_ref, degp_ref, h_ref, q_ref, qb_ref,
                                w_ref, wb_ref, d)

  return pl.pallas_call(
      body,
      grid=(grid,),
      in_specs=[
          pl.BlockSpec((NC, bn, d), lambda i: (0, i, 0)),
          pl.BlockSpec((NW, 1, 1, bn), lambda i: (0, i, 0, 0)),
          pl.BlockSpec((bn, d), lambda i: (i, 0)),
          pl.BlockSpec((d, d), lambda i: (0, 0)),
          pl.BlockSpec((d,), lambda i: (0,)),
          pl.BlockSpec((2 * d, d), lambda i: (0, 0)),
          pl.BlockSpec((d,), lambda i: (0,)),
      ],
      out_specs=pl.BlockSpec((bn, d), lambda i: (i, 0)),
      out_shape=jax.ShapeDtypeStruct((n_nodes, d), jnp.float32),
  )(aggp, degp.reshape(NW, grid, 1, bn), h, q, qb, w, wb)


def _tc_layer2_head(aggp, degp, h1, q, qb, w, wb, g_mat, gb,
                    gam1, bet1, gam2, bet2):
  n_nodes, d = h1.shape
  bn = 1000
  grid = n_nodes // bn

  def body(aggp_ref, degp_ref, h1_ref, q_ref, qb_ref, w_ref, wb_ref,
           gm_ref, gb_ref, g1_ref, b1_ref, g2_ref, b2_ref, o_ref, zs_ref):
    i = pl.program_id(0)
    h2 = _pinconv_block(aggp_ref, degp_ref, h1_ref, q_ref, qb_ref,
                        w_ref, wb_ref, d)
    z = jnp.maximum(
        lax.dot_general(h2, gm_ref[...], (((1,), (0,)), ((), ())),
                        preferred_element_type=jnp.float32) + gb_ref[...],
        0.0)
    zs_ref[pl.ds(i * bn, bn), :] = z

    @pl.when(i == grid - 1)
    def _():
      zz = zs_ref[...]
      inv_n = 1.0 / n_nodes
      mu = jnp.sum(zz, axis=0) * inv_n
      c = zz - mu
      var = jnp.sum(c * c, axis=0) * inv_n
      a1 = g1_ref[...] / jnp.sqrt(var + 1e-5)
      a2 = g2_ref[...] / jnp.sqrt(a1 * a1 * var + 1e-5)
      o_ref[...] = (a2 * a1) * c + b2_ref[...]

  vec = pl.BlockSpec((d,), lambda i: (0,))
  return pl.pallas_call(
      body,
      grid=(grid,),
      in_specs=[
          pl.BlockSpec((NC, bn, d), lambda i: (0, i, 0)),
          pl.BlockSpec((NW, 1, 1, bn), lambda i: (0, i, 0, 0)),
          pl.BlockSpec((bn, d), lambda i: (i, 0)),
          pl.BlockSpec((d, d), lambda i: (0, 0)),
          vec,
          pl.BlockSpec((2 * d, d), lambda i: (0, 0)),
          vec,
          pl.BlockSpec((d, d), lambda i: (0, 0)),
          vec, vec, vec, vec, vec,
      ],
      out_specs=pl.BlockSpec((n_nodes, d), lambda i: (0, 0)),
      out_shape=jax.ShapeDtypeStruct((n_nodes, d), jnp.float32),
      scratch_shapes=[pltpu.VMEM((n_nodes, d), jnp.float32)],
  )(aggp, degp.reshape(NW, grid, 1, bn), h1, q, qb, w, wb, g_mat, gb,
    gam1, bet1, gam2, bet2)


def kernel(h, edge_index, Q0, qb0, W0, wb0, Q1, qb1, W1, wb1, G_mat, Gb, g,
           bn_out_gamma, bn_out_beta, bn_gamma, bn_beta):
  e = edge_index.shape[1]
  d = h.shape[1]
  edges3d = edge_index.reshape(2, e // G, G)
  eflat = edge_index.reshape(2 * e)
  zeros_gd = jnp.zeros((640, d), jnp.float32)

  agg1p, degp = _sc_aggregate(h, edges3d, eflat, zeros_gd, with_deg=True)
  h1 = _tc_layer(agg1p, degp, h, Q0, qb0, W0, wb0)
  (agg2p,) = _sc_aggregate(h1, edges3d, eflat, zeros_gd, with_deg=False)
  gam1 = g * bn_out_gamma
  bet1 = g * bn_out_beta
  return _tc_layer2_head(agg2p, degp, h1, Q1, qb1, W1, wb1, G_mat, Gb,
                         gam1, bet1, bn_gamma, bn_beta)
